# Initial kernel scaffold; baseline (speedup 1.0000x reference)
import jax
import jax.numpy as jnp
from jax.experimental import pallas as pl


def kernel(pos_1, pos_2, table1, table2):
    raise NotImplementedError("write your pallas kernel here")


# speedup vs baseline: 4.8949x; 4.8949x over previous
import functools

import jax
import jax.numpy as jnp
from jax import lax
from jax.experimental import pallas as pl
from jax.experimental.pallas import tpu as pltpu
from jax.experimental.pallas import tpu_sc as plsc

B, L, D, V = 4096, 200, 64, 1024
N = B * L
NC, NS = 2, 16
NW = NC * NS
PER_W = N // NW
C = 128
NIT = PER_W // C

_mesh = plsc.VectorSubcoreMesh(core_axis_name="c", subcore_axis_name="s")


@functools.partial(
    pl.kernel,
    mesh=_mesh,
    compiler_params=pltpu.CompilerParams(use_tc_tiling_on_sc=False),
    out_type=jax.ShapeDtypeStruct((N, D), jnp.float32),
    scratch_types=[
        pltpu.VMEM((C,), jnp.int32),
        pltpu.VMEM((C,), jnp.int32),
        pltpu.VMEM((C, D), jnp.float32),
        pltpu.VMEM((C, D), jnp.float32),
        pltpu.SemaphoreType.DMA,
        pltpu.SemaphoreType.DMA,
    ],
)
def _emb_sum_kernel(i1_hbm, i2_hbm, t1_hbm, t2_hbm, out_hbm,
                    idx1, idx2, r1, r2, sem1, sem2):
    wid = lax.axis_index("s") * NC + lax.axis_index("c")
    base = wid * PER_W

    def body(it, carry):
        off = base + it * C
        pltpu.sync_copy(i1_hbm.at[pl.ds(off, C)], idx1)
        pltpu.sync_copy(i2_hbm.at[pl.ds(off, C)], idx2)
        cp1 = pltpu.async_copy(t1_hbm.at[idx1], r1, sem1)
        cp2 = pltpu.async_copy(t2_hbm.at[idx2], r2, sem2)
        cp1.wait()
        cp2.wait()

        def add_rows(i, c2):
            for u in range(8):
                for col in range(D // 16):
                    s = pl.ds(col * 16, 16)
                    r1[i * 8 + u, s] = r1[i * 8 + u, s] + r2[i * 8 + u, s]
            return c2

        lax.fori_loop(0, C // 8, add_rows, 0)
        pltpu.sync_copy(r1, out_hbm.at[pl.ds(off, C)])
        return carry

    lax.fori_loop(0, NIT, body, 0)


def kernel(pos_1, pos_2, table1, table2):
    zrow = jnp.zeros((1, D), jnp.float32)
    t1a = jnp.concatenate([zrow, table1[: V - 1]], axis=0)
    t2a = jnp.concatenate([zrow, table2[: V - 1]], axis=0)
    i1 = pos_1.reshape(N).astype(jnp.int32)
    i2 = pos_2.reshape(N).astype(jnp.int32)
    out = _emb_sum_kernel(i1, i2, t1a, t2a)
    return out.reshape(B, L, D)
